# baseline (speedup 1.0000x reference)
import functools

import jax
import jax.numpy as jnp
from jax import lax
from jax.experimental import pallas as pl
from jax.experimental.pallas import tpu as pltpu
from jax.experimental.pallas import tpu_sc as plsc

B = 8
H = 512
W = 512
N = H * W
NTOT = B * N
NW = 32
WPS = NW // B
ROWS_W = H // WPS
RCH = 32
NCH = ROWS_W // RCH
VPR = W // 16

_C0 = 0.0009251831215806305
_C1 = 0.9797525405883789
_C2 = -0.3935345709323883
_C3 = 0.10668430477380753
def _softplus_nc(xv):
    e = jnp.exp(-jnp.abs(xv))
    p = (jnp.float32(_C3) * e + jnp.float32(_C2)) * e + jnp.float32(_C1)
    return jnp.maximum(xv, jnp.float32(0.0)) + p * e


_MESH = plsc.VectorSubcoreMesh(core_axis_name="c", subcore_axis_name="s",
                               num_cores=2, num_subcores=16)


def _sc_sums_body(x_hbm, out_hbm, xb0, xb1, ob, sx0, sx1):
    c = lax.axis_index("c")
    s = lax.axis_index("s")
    wid = s * 2 + c
    samp = wid // WPS
    row0 = (wid % WPS) * ROWS_W
    xbufs = (xb0, xb1)
    sxs = (sx0, sx1)

    def start(ci):
        r = row0 + ci * RCH
        b = ci & 1
        return pltpu.async_copy(x_hbm.at[samp, 0, pl.ds(r, RCH), :],
                                xbufs[b], sxs[b])

    def compute_chunk(xb, acc0):
        def row_loop(ri, carry2):
            def vloop(i, al):
                xv = xb[ri, pl.ds(i * 16, 16)]
                return al + _softplus_nc(xv)

            return lax.fori_loop(0, VPR, vloop, carry2, unroll=8)

        return lax.fori_loop(0, RCH, row_loop, acc0)

    acc = jnp.zeros((16,), jnp.float32)
    h = start(0)
    for ci in range(NCH):
        hx = h
        if ci + 1 < NCH:
            h = start(ci + 1)
        hx.wait()
        acc = compute_chunk(xbufs[ci & 1], acc)
    ob[0, :] = acc
    pltpu.sync_copy(ob, out_hbm.at[wid])


RRCH = 32
RNCH = H // RRCH


def _key_u32(xv, tv):
    b = lax.bitcast_convert_type(xv, jnp.uint32)
    neg = (b >> jnp.uint32(31)) == jnp.uint32(1)
    key = jnp.where(neg, ~b, b | jnp.uint32(0x80000000))
    key = jnp.maximum(key, jnp.uint32(1))
    return jnp.where(tv == jnp.float32(1.0), jnp.uint32(0), key)


def _lane_sum(v):
    idx = lax.iota(jnp.int32, 16)
    for sh in (1, 2, 4, 8):
        rot = v.at[(idx + sh) & 15].get(mode="promise_in_bounds")
        v = v + rot
    return v


def _sc_topk_body(x_hbm, t_hbm, out_hbm, xb, tb, ob):
    c = lax.axis_index("c")
    s = lax.axis_index("s")
    wid = s * 2 + c
    samp = jnp.minimum(wid, B - 1)
    zi = jnp.zeros((16,), jnp.int32)
    zf = jnp.zeros((16,), jnp.float32)

    def count_pass(cand_incl):
        def chunk_loop(ci, acc):
            pltpu.sync_copy(x_hbm.at[samp, 0, pl.ds(ci * RRCH, RRCH), :], xb)
            pltpu.sync_copy(t_hbm.at[samp, 0, pl.ds(ci * RRCH, RRCH), :], tb)

            def row_loop(ri, a0):
                def vloop(i, a):
                    key = _key_u32(xb[ri, pl.ds(i * 16, 16)],
                                   tb[ri, pl.ds(i * 16, 16)])
                    return a + jnp.where(key >= cand_incl, jnp.int32(1),
                                         jnp.int32(0))

                return lax.fori_loop(0, VPR, vloop, a0, unroll=4)

            return lax.fori_loop(0, RRCH, row_loop, acc)

        acc = lax.fori_loop(0, RNCH, chunk_loop, zi)
        return _lane_sum(acc)

    def nw_chunk(ci, acc):
        pltpu.sync_copy(t_hbm.at[samp, 0, pl.ds(ci * RRCH, RRCH), :], tb)

        def row_loop(ri, a0):
            def vloop(i, a):
                return a + tb[ri, pl.ds(i * 16, 16)]

            return lax.fori_loop(0, VPR, vloop, a0, unroll=4)

        return lax.fori_loop(0, RRCH, row_loop, acc)

    nwv = lax.fori_loop(0, RNCH, nw_chunk, zf)
    n_white_f = _lane_sum(nwv)
    n_white = n_white_f.astype(jnp.int32)
    n_black = jnp.full((16,), N, jnp.int32) - n_white
    k = jnp.minimum(3 * n_white, n_black)

    def bit_step(j, prefix):
        bit = jnp.full((16,), 1, jnp.uint32) << (
            jnp.uint32(31) - j.astype(jnp.uint32))
        cand = prefix | bit
        cnt = count_pass(cand)
        return jnp.where(cnt >= k, cand, prefix)

    T = lax.fori_loop(0, 32, bit_step, jnp.zeros((16,), jnp.uint32))

    def fin_chunk(ci, carry):
        pltpu.sync_copy(x_hbm.at[samp, 0, pl.ds(ci * RRCH, RRCH), :], xb)
        pltpu.sync_copy(t_hbm.at[samp, 0, pl.ds(ci * RRCH, RRCH), :], tb)

        def row_loop(ri, c0):
            def vloop(i, cc):
                aw, abs_, abc = cc
                xv = xb[ri, pl.ds(i * 16, 16)]
                tv = tb[ri, pl.ds(i * 16, 16)]
                sp = _softplus_nc(xv) + jnp.float32(_C0)
                key = _key_u32(xv, tv)
                white = tv == jnp.float32(1.0)
                gt = key > T
                aw = aw + jnp.where(white, sp - xv, jnp.float32(0.0))
                abs_ = abs_ + jnp.where(gt, sp, jnp.float32(0.0))
                abc = abc + jnp.where(gt, jnp.float32(1.0), jnp.float32(0.0))
                return aw, abs_, abc

            return lax.fori_loop(0, VPR, vloop, c0, unroll=4)

        return lax.fori_loop(0, RRCH, row_loop, carry)

    aw, abs_, abc = lax.fori_loop(0, RNCH, fin_chunk, (zf, zf, zf))
    sum_white = _lane_sum(aw)
    sum_gt = _lane_sum(abs_)
    cnt_gt = _lane_sum(abc)

    tbits = jnp.where(T >= jnp.uint32(0x80000000), T & jnp.uint32(0x7FFFFFFF),
                      ~T)
    sp_tie = _softplus_nc(lax.bitcast_convert_type(tbits, jnp.float32)) + \
        jnp.float32(_C0)
    n_tie = k.astype(jnp.float32) - cnt_gt
    sum_black = sum_gt + jnp.where(k > 0, n_tie * sp_tie, jnp.float32(0.0))

    ob[0, :] = sum_white + sum_black
    ob[1, :] = n_white_f + k.astype(jnp.float32)
    ob[2, :] = zf
    ob[3, :] = zf
    pltpu.sync_copy(ob, out_hbm.at[wid])


def _build_kernels(interpret=False):
    sums = pl.kernel(
        _sc_sums_body,
        out_type=jax.ShapeDtypeStruct((NW, 1, 16), jnp.float32),
        mesh=_MESH,
        scratch_types=[
            pltpu.VMEM((RCH, W), jnp.float32),
            pltpu.VMEM((RCH, W), jnp.float32),
            pltpu.VMEM((1, 16), jnp.float32),
            pltpu.SemaphoreType.DMA,
            pltpu.SemaphoreType.DMA,
        ],
        interpret=interpret,
    )
    topk = pl.kernel(
        _sc_topk_body,
        out_type=jax.ShapeDtypeStruct((NW, 4, 16), jnp.float32),
        mesh=_MESH,
        scratch_types=[
            pltpu.VMEM((RRCH, W), jnp.float32),
            pltpu.VMEM((RRCH, W), jnp.float32),
            pltpu.VMEM((4, 16), jnp.float32),
        ],
        interpret=interpret,
    )
    return sums, topk


_sc_sums, _sc_topk = _build_kernels()


def kernel(input, target):
    parts = _sc_sums(input)
    tw = jnp.sum(target.reshape(B, N), axis=1)
    xw = jnp.sum((input * target).reshape(B, N))
    sp_sum = jnp.sum(parts) + jnp.float32(NTOT * _C0)
    loss_sum = sp_sum - xw
    n_white = tw
    n_black = jnp.float32(N) - n_white
    common = loss_sum / jnp.float32(NTOT)

    def rare():
        out = _sc_topk(input, target)
        sums = out[:B, 0, 0]
        cnts = out[:B, 1, 0]
        return jnp.sum(sums) / jnp.sum(cnts)

    pred = jnp.all(3.0 * n_white >= n_black)
    return lax.cond(pred, lambda: common, rare)


# speedup vs baseline: 1.1967x; 1.1967x over previous
import functools

import jax
import jax.numpy as jnp
from jax import lax
from jax.experimental import pallas as pl
from jax.experimental.pallas import tpu as pltpu
from jax.experimental.pallas import tpu_sc as plsc

B = 8
H = 512
W = 512
N = H * W
NTOT = B * N
NW = 32
WPS = NW // B
ROWS_W = H // WPS
RCH = 32
NCH = ROWS_W // RCH
VPR = W // 16

_C0 = 0.0009251831215806305
_C1 = 0.9797525405883789
_C2 = -0.3935345709323883
_C3 = 0.10668430477380753
def _softplus_nc(xv):
    e = jnp.exp(-jnp.abs(xv))
    p = (jnp.float32(_C3) * e + jnp.float32(_C2)) * e + jnp.float32(_C1)
    return jnp.maximum(xv, jnp.float32(0.0)) + p * e


_MESH = plsc.VectorSubcoreMesh(core_axis_name="c", subcore_axis_name="s",
                               num_cores=2, num_subcores=16)


def _sc_sums_body(x_hbm, out_hbm, xb0, xb1, ob, sx0, sx1):
    c = lax.axis_index("c")
    s = lax.axis_index("s")
    wid = s * 2 + c
    samp = wid // WPS
    row0 = (wid % WPS) * ROWS_W
    xbufs = (xb0, xb1)
    sxs = (sx0, sx1)

    def start(ci):
        r = row0 + ci * RCH
        b = ci & 1
        return pltpu.async_copy(x_hbm.at[samp, 0, pl.ds(r, RCH), :],
                                xbufs[b], sxs[b])

    def compute_chunk(xb, acc0):
        def row_loop(ri, carry2):
            def vloop(i, al):
                xv = xb[ri, pl.ds(i * 16, 16)]
                return al + _softplus_nc(xv)

            return lax.fori_loop(0, VPR, vloop, carry2, unroll=8)

        return lax.fori_loop(0, RCH, row_loop, acc0)

    acc = jnp.zeros((16,), jnp.float32)
    h = start(0)
    for ci in range(NCH):
        hx = h
        if ci + 1 < NCH:
            h = start(ci + 1)
        hx.wait()
        acc = compute_chunk(xbufs[ci & 1], acc)
    ob[0, :] = acc
    pltpu.sync_copy(ob, out_hbm.at[wid])


RRCH = 32
RNCH = H // RRCH


def _key_u32(xv, tv):
    b = lax.bitcast_convert_type(xv, jnp.uint32)
    neg = (b >> jnp.uint32(31)) == jnp.uint32(1)
    key = jnp.where(neg, ~b, b | jnp.uint32(0x80000000))
    key = jnp.maximum(key, jnp.uint32(1))
    return jnp.where(tv == jnp.float32(1.0), jnp.uint32(0), key)


def _lane_sum(v):
    idx = lax.iota(jnp.int32, 16)
    for sh in (1, 2, 4, 8):
        rot = v.at[(idx + sh) & 15].get(mode="promise_in_bounds")
        v = v + rot
    return v


def _sc_topk_body(x_hbm, t_hbm, out_hbm, xb, tb, ob):
    c = lax.axis_index("c")
    s = lax.axis_index("s")
    wid = s * 2 + c
    samp = jnp.minimum(wid, B - 1)
    zi = jnp.zeros((16,), jnp.int32)
    zf = jnp.zeros((16,), jnp.float32)

    def count_pass(cand_incl):
        def chunk_loop(ci, acc):
            pltpu.sync_copy(x_hbm.at[samp, 0, pl.ds(ci * RRCH, RRCH), :], xb)
            pltpu.sync_copy(t_hbm.at[samp, 0, pl.ds(ci * RRCH, RRCH), :], tb)

            def row_loop(ri, a0):
                def vloop(i, a):
                    key = _key_u32(xb[ri, pl.ds(i * 16, 16)],
                                   tb[ri, pl.ds(i * 16, 16)])
                    return a + jnp.where(key >= cand_incl, jnp.int32(1),
                                         jnp.int32(0))

                return lax.fori_loop(0, VPR, vloop, a0, unroll=4)

            return lax.fori_loop(0, RRCH, row_loop, acc)

        acc = lax.fori_loop(0, RNCH, chunk_loop, zi)
        return _lane_sum(acc)

    def nw_chunk(ci, acc):
        pltpu.sync_copy(t_hbm.at[samp, 0, pl.ds(ci * RRCH, RRCH), :], tb)

        def row_loop(ri, a0):
            def vloop(i, a):
                return a + tb[ri, pl.ds(i * 16, 16)]

            return lax.fori_loop(0, VPR, vloop, a0, unroll=4)

        return lax.fori_loop(0, RRCH, row_loop, acc)

    nwv = lax.fori_loop(0, RNCH, nw_chunk, zf)
    n_white_f = _lane_sum(nwv)
    n_white = n_white_f.astype(jnp.int32)
    n_black = jnp.full((16,), N, jnp.int32) - n_white
    k = jnp.minimum(3 * n_white, n_black)

    def bit_step(j, prefix):
        bit = jnp.full((16,), 1, jnp.uint32) << (
            jnp.uint32(31) - j.astype(jnp.uint32))
        cand = prefix | bit
        cnt = count_pass(cand)
        return jnp.where(cnt >= k, cand, prefix)

    T = lax.fori_loop(0, 32, bit_step, jnp.zeros((16,), jnp.uint32))

    def fin_chunk(ci, carry):
        pltpu.sync_copy(x_hbm.at[samp, 0, pl.ds(ci * RRCH, RRCH), :], xb)
        pltpu.sync_copy(t_hbm.at[samp, 0, pl.ds(ci * RRCH, RRCH), :], tb)

        def row_loop(ri, c0):
            def vloop(i, cc):
                aw, abs_, abc = cc
                xv = xb[ri, pl.ds(i * 16, 16)]
                tv = tb[ri, pl.ds(i * 16, 16)]
                sp = _softplus_nc(xv) + jnp.float32(_C0)
                key = _key_u32(xv, tv)
                white = tv == jnp.float32(1.0)
                gt = key > T
                aw = aw + jnp.where(white, sp - xv, jnp.float32(0.0))
                abs_ = abs_ + jnp.where(gt, sp, jnp.float32(0.0))
                abc = abc + jnp.where(gt, jnp.float32(1.0), jnp.float32(0.0))
                return aw, abs_, abc

            return lax.fori_loop(0, VPR, vloop, c0, unroll=4)

        return lax.fori_loop(0, RRCH, row_loop, carry)

    aw, abs_, abc = lax.fori_loop(0, RNCH, fin_chunk, (zf, zf, zf))
    sum_white = _lane_sum(aw)
    sum_gt = _lane_sum(abs_)
    cnt_gt = _lane_sum(abc)

    tbits = jnp.where(T >= jnp.uint32(0x80000000), T & jnp.uint32(0x7FFFFFFF),
                      ~T)
    sp_tie = _softplus_nc(lax.bitcast_convert_type(tbits, jnp.float32)) + \
        jnp.float32(_C0)
    n_tie = k.astype(jnp.float32) - cnt_gt
    sum_black = sum_gt + jnp.where(k > 0, n_tie * sp_tie, jnp.float32(0.0))

    ob[0, :] = sum_white + sum_black
    ob[1, :] = n_white_f + k.astype(jnp.float32)
    ob[2, :] = zf
    ob[3, :] = zf
    pltpu.sync_copy(ob, out_hbm.at[wid])


def _build_kernels(interpret=False):
    sums = pl.kernel(
        _sc_sums_body,
        out_type=jax.ShapeDtypeStruct((NW, 1, 16), jnp.float32),
        mesh=_MESH,
        scratch_types=[
            pltpu.VMEM((RCH, W), jnp.float32),
            pltpu.VMEM((RCH, W), jnp.float32),
            pltpu.VMEM((1, 16), jnp.float32),
            pltpu.SemaphoreType.DMA,
            pltpu.SemaphoreType.DMA,
        ],
        interpret=interpret,
    )
    topk = pl.kernel(
        _sc_topk_body,
        out_type=jax.ShapeDtypeStruct((NW, 4, 16), jnp.float32),
        mesh=_MESH,
        scratch_types=[
            pltpu.VMEM((RRCH, W), jnp.float32),
            pltpu.VMEM((RRCH, W), jnp.float32),
            pltpu.VMEM((4, 16), jnp.float32),
        ],
        interpret=interpret,
    )
    return sums, topk


_sc_sums, _sc_topk = _build_kernels()


def kernel(input, target):
    parts = _sc_sums(input)
    tw = jnp.sum(target.reshape(B, N), axis=1)
    xw = jnp.sum((input * target).reshape(B, N))
    sp_sum = jnp.sum(parts) + jnp.float32(NTOT * _C0)
    loss_sum = sp_sum - xw
    n_white = tw
    n_black = jnp.float32(N) - n_white
    common = loss_sum / jnp.float32(NTOT)

    def rare():
        out = _sc_topk(input, target)
        sums = out[:B, 0, 0]
        cnts = out[:B, 1, 0]
        return jnp.sum(sums) / jnp.sum(cnts)

    pred = jnp.all(3.0 * n_white >= n_black) & jnp.isfinite(xw)
    return lax.cond(pred, lambda: common, rare)


# speedup vs baseline: 1.2634x; 1.0558x over previous
import functools

import jax
import jax.numpy as jnp
from jax import lax
from jax.experimental import pallas as pl
from jax.experimental.pallas import tpu as pltpu
from jax.experimental.pallas import tpu_sc as plsc

B = 8
H = 512
W = 512
N = H * W
NTOT = B * N
NW = 32
WPS = NW // B
R_SC = 384
ROWS_W = R_SC // WPS
RCH = 32
NCH = ROWS_W // RCH
VPR = W // 16

_C0 = 0.0009251831215806305
_C1 = 0.9797525405883789
_C2 = -0.3935345709323883
_C3 = 0.10668430477380753
def _softplus_nc(xv):
    e = jnp.exp(-jnp.abs(xv))
    p = (jnp.float32(_C3) * e + jnp.float32(_C2)) * e + jnp.float32(_C1)
    return jnp.maximum(xv, jnp.float32(0.0)) + p * e


_MESH = plsc.VectorSubcoreMesh(core_axis_name="c", subcore_axis_name="s",
                               num_cores=2, num_subcores=16)


def _sc_sums_body(x_hbm, out_hbm, xb0, xb1, ob, sx0, sx1):
    c = lax.axis_index("c")
    s = lax.axis_index("s")
    wid = s * 2 + c
    samp = wid // WPS
    row0 = (wid % WPS) * ROWS_W
    xbufs = (xb0, xb1)
    sxs = (sx0, sx1)

    def start(ci):
        r = row0 + ci * RCH
        b = ci & 1
        return pltpu.async_copy(x_hbm.at[samp, 0, pl.ds(r, RCH), :],
                                xbufs[b], sxs[b])

    def compute_chunk(xb, acc0):
        def row_loop(ri, carry2):
            def vloop(i, al):
                xv = xb[ri, pl.ds(i * 16, 16)]
                return al + _softplus_nc(xv)

            return lax.fori_loop(0, VPR, vloop, carry2, unroll=8)

        return lax.fori_loop(0, RCH, row_loop, acc0)

    acc = jnp.zeros((16,), jnp.float32)
    h = start(0)
    for ci in range(NCH):
        hx = h
        if ci + 1 < NCH:
            h = start(ci + 1)
        hx.wait()
        acc = compute_chunk(xbufs[ci & 1], acc)
    ob[0, :] = acc
    pltpu.sync_copy(ob, out_hbm.at[wid])


RRCH = 32
RNCH = H // RRCH


def _key_u32(xv, tv):
    b = lax.bitcast_convert_type(xv, jnp.uint32)
    neg = (b >> jnp.uint32(31)) == jnp.uint32(1)
    key = jnp.where(neg, ~b, b | jnp.uint32(0x80000000))
    key = jnp.maximum(key, jnp.uint32(1))
    return jnp.where(tv == jnp.float32(1.0), jnp.uint32(0), key)


def _lane_sum(v):
    idx = lax.iota(jnp.int32, 16)
    for sh in (1, 2, 4, 8):
        rot = v.at[(idx + sh) & 15].get(mode="promise_in_bounds")
        v = v + rot
    return v


def _sc_topk_body(x_hbm, t_hbm, out_hbm, xb, tb, ob):
    c = lax.axis_index("c")
    s = lax.axis_index("s")
    wid = s * 2 + c
    samp = jnp.minimum(wid, B - 1)
    zi = jnp.zeros((16,), jnp.int32)
    zf = jnp.zeros((16,), jnp.float32)

    def count_pass(cand_incl):
        def chunk_loop(ci, acc):
            pltpu.sync_copy(x_hbm.at[samp, 0, pl.ds(ci * RRCH, RRCH), :], xb)
            pltpu.sync_copy(t_hbm.at[samp, 0, pl.ds(ci * RRCH, RRCH), :], tb)

            def row_loop(ri, a0):
                def vloop(i, a):
                    key = _key_u32(xb[ri, pl.ds(i * 16, 16)],
                                   tb[ri, pl.ds(i * 16, 16)])
                    return a + jnp.where(key >= cand_incl, jnp.int32(1),
                                         jnp.int32(0))

                return lax.fori_loop(0, VPR, vloop, a0, unroll=4)

            return lax.fori_loop(0, RRCH, row_loop, acc)

        acc = lax.fori_loop(0, RNCH, chunk_loop, zi)
        return _lane_sum(acc)

    def nw_chunk(ci, acc):
        pltpu.sync_copy(t_hbm.at[samp, 0, pl.ds(ci * RRCH, RRCH), :], tb)

        def row_loop(ri, a0):
            def vloop(i, a):
                return a + tb[ri, pl.ds(i * 16, 16)]

            return lax.fori_loop(0, VPR, vloop, a0, unroll=4)

        return lax.fori_loop(0, RRCH, row_loop, acc)

    nwv = lax.fori_loop(0, RNCH, nw_chunk, zf)
    n_white_f = _lane_sum(nwv)
    n_white = n_white_f.astype(jnp.int32)
    n_black = jnp.full((16,), N, jnp.int32) - n_white
    k = jnp.minimum(3 * n_white, n_black)

    def bit_step(j, prefix):
        bit = jnp.full((16,), 1, jnp.uint32) << (
            jnp.uint32(31) - j.astype(jnp.uint32))
        cand = prefix | bit
        cnt = count_pass(cand)
        return jnp.where(cnt >= k, cand, prefix)

    T = lax.fori_loop(0, 32, bit_step, jnp.zeros((16,), jnp.uint32))

    def fin_chunk(ci, carry):
        pltpu.sync_copy(x_hbm.at[samp, 0, pl.ds(ci * RRCH, RRCH), :], xb)
        pltpu.sync_copy(t_hbm.at[samp, 0, pl.ds(ci * RRCH, RRCH), :], tb)

        def row_loop(ri, c0):
            def vloop(i, cc):
                aw, abs_, abc = cc
                xv = xb[ri, pl.ds(i * 16, 16)]
                tv = tb[ri, pl.ds(i * 16, 16)]
                sp = _softplus_nc(xv) + jnp.float32(_C0)
                key = _key_u32(xv, tv)
                white = tv == jnp.float32(1.0)
                gt = key > T
                aw = aw + jnp.where(white, sp - xv, jnp.float32(0.0))
                abs_ = abs_ + jnp.where(gt, sp, jnp.float32(0.0))
                abc = abc + jnp.where(gt, jnp.float32(1.0), jnp.float32(0.0))
                return aw, abs_, abc

            return lax.fori_loop(0, VPR, vloop, c0, unroll=4)

        return lax.fori_loop(0, RRCH, row_loop, carry)

    aw, abs_, abc = lax.fori_loop(0, RNCH, fin_chunk, (zf, zf, zf))
    sum_white = _lane_sum(aw)
    sum_gt = _lane_sum(abs_)
    cnt_gt = _lane_sum(abc)

    tbits = jnp.where(T >= jnp.uint32(0x80000000), T & jnp.uint32(0x7FFFFFFF),
                      ~T)
    sp_tie = _softplus_nc(lax.bitcast_convert_type(tbits, jnp.float32)) + \
        jnp.float32(_C0)
    n_tie = k.astype(jnp.float32) - cnt_gt
    sum_black = sum_gt + jnp.where(k > 0, n_tie * sp_tie, jnp.float32(0.0))

    ob[0, :] = sum_white + sum_black
    ob[1, :] = n_white_f + k.astype(jnp.float32)
    ob[2, :] = zf
    ob[3, :] = zf
    pltpu.sync_copy(ob, out_hbm.at[wid])


def _tc_part_body(x_ref, t_ref, o_ref):
    xs = x_ref[0, 0]
    ts = t_ref[0, 0]
    tw = jnp.sum(ts)
    xw = jnp.sum(xs * ts)
    xhi = xs[R_SC:, :]
    sp = jnp.sum(jnp.maximum(xhi, 0.0) + jnp.log1p(jnp.exp(-jnp.abs(xhi))))
    o_ref[0] = jnp.stack([jnp.full((128,), tw, jnp.float32),
                          jnp.full((128,), xw, jnp.float32),
                          jnp.full((128,), sp, jnp.float32)])


_tc_part = pl.pallas_call(
    _tc_part_body,
    grid=(B,),
    in_specs=[
        pl.BlockSpec((1, 1, H, W), lambda b: (b, 0, 0, 0)),
        pl.BlockSpec((1, 1, H, W), lambda b: (b, 0, 0, 0)),
    ],
    out_specs=pl.BlockSpec((1, 3, 128), lambda b: (b, 0, 0)),
    out_shape=jax.ShapeDtypeStruct((B, 3, 128), jnp.float32),
)


def _build_kernels(interpret=False):
    sums = pl.kernel(
        _sc_sums_body,
        out_type=jax.ShapeDtypeStruct((NW, 1, 16), jnp.float32),
        mesh=_MESH,
        scratch_types=[
            pltpu.VMEM((RCH, W), jnp.float32),
            pltpu.VMEM((RCH, W), jnp.float32),
            pltpu.VMEM((1, 16), jnp.float32),
            pltpu.SemaphoreType.DMA,
            pltpu.SemaphoreType.DMA,
        ],
        interpret=interpret,
    )
    topk = pl.kernel(
        _sc_topk_body,
        out_type=jax.ShapeDtypeStruct((NW, 4, 16), jnp.float32),
        mesh=_MESH,
        scratch_types=[
            pltpu.VMEM((RRCH, W), jnp.float32),
            pltpu.VMEM((RRCH, W), jnp.float32),
            pltpu.VMEM((4, 16), jnp.float32),
        ],
        interpret=interpret,
    )
    return sums, topk


_sc_sums, _sc_topk = _build_kernels()


def kernel(input, target):
    parts = _sc_sums(input)
    tc = _tc_part(input, target)
    tw = tc[:, 0, 0]
    xw = jnp.sum(tc[:, 1, 0])
    sp_hi = jnp.sum(tc[:, 2, 0])
    sp_sum = jnp.sum(parts) + jnp.float32(B * R_SC * W * _C0) + sp_hi
    loss_sum = sp_sum - xw
    n_white = tw
    n_black = jnp.float32(N) - n_white
    common = loss_sum / jnp.float32(NTOT)

    def rare():
        out = _sc_topk(input, target)
        sums = out[:B, 0, 0]
        cnts = out[:B, 1, 0]
        return jnp.sum(sums) / jnp.sum(cnts)

    pred = jnp.all(3.0 * n_white >= n_black) & jnp.isfinite(xw)
    return lax.cond(pred, lambda: common, rare)


# speedup vs baseline: 1.3188x; 1.0439x over previous
import functools

import jax
import jax.numpy as jnp
from jax import lax
from jax.experimental import pallas as pl
from jax.experimental.pallas import tpu as pltpu
from jax.experimental.pallas import tpu_sc as plsc

B = 8
H = 512
W = 512
N = H * W
NTOT = B * N
NW = 32
WPS = NW // B
R_SC = 320
ROWS_W = R_SC // WPS
RCH = 40
NCH = ROWS_W // RCH
VPR = W // 16

_C0 = 0.0009251831215806305
_C1 = 0.9797525405883789
_C2 = -0.3935345709323883
_C3 = 0.10668430477380753
def _softplus_nc(xv):
    e = jnp.exp(-jnp.abs(xv))
    p = (jnp.float32(_C3) * e + jnp.float32(_C2)) * e + jnp.float32(_C1)
    return jnp.maximum(xv, jnp.float32(0.0)) + p * e


_MESH = plsc.VectorSubcoreMesh(core_axis_name="c", subcore_axis_name="s",
                               num_cores=2, num_subcores=16)


def _sc_sums_body(x_hbm, out_hbm, xb0, xb1, ob, sx0, sx1):
    c = lax.axis_index("c")
    s = lax.axis_index("s")
    wid = s * 2 + c
    samp = wid // WPS
    row0 = (wid % WPS) * ROWS_W
    xbufs = (xb0, xb1)
    sxs = (sx0, sx1)

    def start(ci):
        r = row0 + ci * RCH
        b = ci & 1
        return pltpu.async_copy(x_hbm.at[samp, 0, pl.ds(r, RCH), :],
                                xbufs[b], sxs[b])

    def compute_chunk(xb, acc0):
        def row_loop(ri, carry2):
            def vloop(i, al):
                xv = xb[ri, pl.ds(i * 16, 16)]
                return al + _softplus_nc(xv)

            return lax.fori_loop(0, VPR, vloop, carry2, unroll=8)

        return lax.fori_loop(0, RCH, row_loop, acc0)

    acc = jnp.zeros((16,), jnp.float32)
    h = start(0)
    for ci in range(NCH):
        hx = h
        if ci + 1 < NCH:
            h = start(ci + 1)
        hx.wait()
        acc = compute_chunk(xbufs[ci & 1], acc)
    ob[0, :] = acc
    pltpu.sync_copy(ob, out_hbm.at[wid])


RRCH = 32
RNCH = H // RRCH


def _key_u32(xv, tv):
    b = lax.bitcast_convert_type(xv, jnp.uint32)
    neg = (b >> jnp.uint32(31)) == jnp.uint32(1)
    key = jnp.where(neg, ~b, b | jnp.uint32(0x80000000))
    key = jnp.maximum(key, jnp.uint32(1))
    return jnp.where(tv == jnp.float32(1.0), jnp.uint32(0), key)


def _lane_sum(v):
    idx = lax.iota(jnp.int32, 16)
    for sh in (1, 2, 4, 8):
        rot = v.at[(idx + sh) & 15].get(mode="promise_in_bounds")
        v = v + rot
    return v


def _sc_topk_body(x_hbm, t_hbm, out_hbm, xb, tb, ob):
    c = lax.axis_index("c")
    s = lax.axis_index("s")
    wid = s * 2 + c
    samp = jnp.minimum(wid, B - 1)
    zi = jnp.zeros((16,), jnp.int32)
    zf = jnp.zeros((16,), jnp.float32)

    def count_pass(cand_incl):
        def chunk_loop(ci, acc):
            pltpu.sync_copy(x_hbm.at[samp, 0, pl.ds(ci * RRCH, RRCH), :], xb)
            pltpu.sync_copy(t_hbm.at[samp, 0, pl.ds(ci * RRCH, RRCH), :], tb)

            def row_loop(ri, a0):
                def vloop(i, a):
                    key = _key_u32(xb[ri, pl.ds(i * 16, 16)],
                                   tb[ri, pl.ds(i * 16, 16)])
                    return a + jnp.where(key >= cand_incl, jnp.int32(1),
                                         jnp.int32(0))

                return lax.fori_loop(0, VPR, vloop, a0, unroll=4)

            return lax.fori_loop(0, RRCH, row_loop, acc)

        acc = lax.fori_loop(0, RNCH, chunk_loop, zi)
        return _lane_sum(acc)

    def nw_chunk(ci, acc):
        pltpu.sync_copy(t_hbm.at[samp, 0, pl.ds(ci * RRCH, RRCH), :], tb)

        def row_loop(ri, a0):
            def vloop(i, a):
                return a + tb[ri, pl.ds(i * 16, 16)]

            return lax.fori_loop(0, VPR, vloop, a0, unroll=4)

        return lax.fori_loop(0, RRCH, row_loop, acc)

    nwv = lax.fori_loop(0, RNCH, nw_chunk, zf)
    n_white_f = _lane_sum(nwv)
    n_white = n_white_f.astype(jnp.int32)
    n_black = jnp.full((16,), N, jnp.int32) - n_white
    k = jnp.minimum(3 * n_white, n_black)

    def bit_step(j, prefix):
        bit = jnp.full((16,), 1, jnp.uint32) << (
            jnp.uint32(31) - j.astype(jnp.uint32))
        cand = prefix | bit
        cnt = count_pass(cand)
        return jnp.where(cnt >= k, cand, prefix)

    T = lax.fori_loop(0, 32, bit_step, jnp.zeros((16,), jnp.uint32))

    def fin_chunk(ci, carry):
        pltpu.sync_copy(x_hbm.at[samp, 0, pl.ds(ci * RRCH, RRCH), :], xb)
        pltpu.sync_copy(t_hbm.at[samp, 0, pl.ds(ci * RRCH, RRCH), :], tb)

        def row_loop(ri, c0):
            def vloop(i, cc):
                aw, abs_, abc = cc
                xv = xb[ri, pl.ds(i * 16, 16)]
                tv = tb[ri, pl.ds(i * 16, 16)]
                sp = _softplus_nc(xv) + jnp.float32(_C0)
                key = _key_u32(xv, tv)
                white = tv == jnp.float32(1.0)
                gt = key > T
                aw = aw + jnp.where(white, sp - xv, jnp.float32(0.0))
                abs_ = abs_ + jnp.where(gt, sp, jnp.float32(0.0))
                abc = abc + jnp.where(gt, jnp.float32(1.0), jnp.float32(0.0))
                return aw, abs_, abc

            return lax.fori_loop(0, VPR, vloop, c0, unroll=4)

        return lax.fori_loop(0, RRCH, row_loop, carry)

    aw, abs_, abc = lax.fori_loop(0, RNCH, fin_chunk, (zf, zf, zf))
    sum_white = _lane_sum(aw)
    sum_gt = _lane_sum(abs_)
    cnt_gt = _lane_sum(abc)

    tbits = jnp.where(T >= jnp.uint32(0x80000000), T & jnp.uint32(0x7FFFFFFF),
                      ~T)
    sp_tie = _softplus_nc(lax.bitcast_convert_type(tbits, jnp.float32)) + \
        jnp.float32(_C0)
    n_tie = k.astype(jnp.float32) - cnt_gt
    sum_black = sum_gt + jnp.where(k > 0, n_tie * sp_tie, jnp.float32(0.0))

    ob[0, :] = sum_white + sum_black
    ob[1, :] = n_white_f + k.astype(jnp.float32)
    ob[2, :] = zf
    ob[3, :] = zf
    pltpu.sync_copy(ob, out_hbm.at[wid])


def _tc_part_body(x_ref, t_ref, o_ref):
    xs = x_ref[0, 0]
    ts = t_ref[0, 0]
    tw = jnp.sum(ts)
    xw = jnp.sum(xs * ts)
    xhi = xs[R_SC:, :]
    sp = jnp.sum(jnp.maximum(xhi, 0.0) + jnp.log1p(jnp.exp(-jnp.abs(xhi))))
    o_ref[0] = jnp.stack([jnp.full((128,), tw, jnp.float32),
                          jnp.full((128,), xw, jnp.float32),
                          jnp.full((128,), sp, jnp.float32)])


_tc_part = pl.pallas_call(
    _tc_part_body,
    grid=(B,),
    in_specs=[
        pl.BlockSpec((1, 1, H, W), lambda b: (b, 0, 0, 0)),
        pl.BlockSpec((1, 1, H, W), lambda b: (b, 0, 0, 0)),
    ],
    out_specs=pl.BlockSpec((1, 3, 128), lambda b: (b, 0, 0)),
    out_shape=jax.ShapeDtypeStruct((B, 3, 128), jnp.float32),
)


def _build_kernels(interpret=False):
    sums = pl.kernel(
        _sc_sums_body,
        out_type=jax.ShapeDtypeStruct((NW, 1, 16), jnp.float32),
        mesh=_MESH,
        scratch_types=[
            pltpu.VMEM((RCH, W), jnp.float32),
            pltpu.VMEM((RCH, W), jnp.float32),
            pltpu.VMEM((1, 16), jnp.float32),
            pltpu.SemaphoreType.DMA,
            pltpu.SemaphoreType.DMA,
        ],
        interpret=interpret,
    )
    topk = pl.kernel(
        _sc_topk_body,
        out_type=jax.ShapeDtypeStruct((NW, 4, 16), jnp.float32),
        mesh=_MESH,
        scratch_types=[
            pltpu.VMEM((RRCH, W), jnp.float32),
            pltpu.VMEM((RRCH, W), jnp.float32),
            pltpu.VMEM((4, 16), jnp.float32),
        ],
        interpret=interpret,
    )
    return sums, topk


_sc_sums, _sc_topk = _build_kernels()


def kernel(input, target):
    parts = _sc_sums(input)
    tc = _tc_part(input, target)
    tw = tc[:, 0, 0]
    xw = jnp.sum(tc[:, 1, 0])
    sp_hi = jnp.sum(tc[:, 2, 0])
    sp_sum = jnp.sum(parts) + jnp.float32(B * R_SC * W * _C0) + sp_hi
    loss_sum = sp_sum - xw
    n_white = tw
    n_black = jnp.float32(N) - n_white
    common = loss_sum / jnp.float32(NTOT)

    def rare():
        out = _sc_topk(input, target)
        sums = out[:B, 0, 0]
        cnts = out[:B, 1, 0]
        return jnp.sum(sums) / jnp.sum(cnts)

    pred = jnp.all(3.0 * n_white >= n_black) & jnp.isfinite(xw)
    return lax.cond(pred, lambda: common, rare)
